# baseline (speedup 1.0000x reference)
import functools

import jax
import jax.numpy as jnp
from jax.experimental import pallas as pl
from jax.experimental.pallas import tpu as pltpu

B_IMGS = 2
HW = 1024
C = 128
N_BANK = 50000
G = 16
LANES = 128
TB = G * LANES
T_STEPS = (N_BANK + TB - 1) // TB
N_PAD = T_STEPS * TB
K = 9
BIG = 3.0e38


def _oems_pairs(n):
    pairs = []

    def merge(lo, n2, r):
        step = r * 2
        if step < n2:
            merge(lo, n2, step)
            merge(lo + r, n2, step)
            for i in range(lo + r, lo + n2 - r, step):
                pairs.append((i, i + r))
        else:
            pairs.append((lo, lo + r))

    def sort_range(lo, hi):
        if (hi - lo) >= 1:
            mid = lo + ((hi - lo) // 2)
            sort_range(lo, mid)
            sort_range(mid + 1, hi)
            merge(lo, hi - lo + 1, 1)

    sort_range(0, n - 1)
    return pairs


_SORT_PAIRS = _oems_pairs(G)


def _pruned_clean_ops(n, keep):
    stages = []
    d = n // 2
    while d >= 1:
        stages.append([(i, i + d)
                       for base in range(0, n, 2 * d)
                       for i in range(base, base + d)])
        d //= 2
    needed = set(range(keep))
    pruned = []
    for ops in reversed(stages):
        sp = []
        new_needed = set()
        for (i, j) in ops:
            lo, hi = i in needed, j in needed
            if lo or hi:
                sp.append((i, j, lo, hi))
                new_needed.add(i)
                new_needed.add(j)
        needed = new_needed
        pruned.append(sp)
    return list(reversed(pruned))


_CLEAN_OPS = _pruned_clean_ops(G, K)
KL = K * LANES


RC = 8


def _scorer_body(fv_ref, bankt_ref, pix_ref, img_ref, d_scr, run_ref):
    t = pl.program_id(1)
    fv = fv_ref[...]
    bankt = bankt_ref[...]

    m2 = jnp.sum(bankt * bankt, axis=0, keepdims=True)
    col = t * TB + jax.lax.broadcasted_iota(jnp.int32, (1, TB), 1)
    m2 = jnp.where(col < N_BANK, m2, BIG)

    d_scr[...] = jnp.dot(fv * jnp.float32(-2.0), bankt,
                         preferred_element_type=jnp.float32)

    @pl.when(t == 0)
    def _init():
        run_ref[...] = jnp.full((HW, KL), BIG, jnp.float32)

    def _chunk(i, _):
        rows = pl.ds(i * RC, RC)
        v = [d_scr[rows, j * LANES:(j + 1) * LANES] + m2[:, j * LANES:(j + 1) * LANES]
             for j in range(G)]
        for (i1, j1) in _SORT_PAIRS:
            lo = jnp.minimum(v[i1], v[j1])
            hi = jnp.maximum(v[i1], v[j1])
            v[i1] = lo
            v[j1] = hi
        r = [run_ref[rows, j * LANES:(j + 1) * LANES] for j in range(K)]
        c = ([jnp.minimum(r[j], v[G - 1 - j]) for j in range(K)]
             + [v[G - 1 - j] for j in range(K, G)])
        for stage in _CLEAN_OPS:
            for (i2, j2, lo_need, hi_need) in stage:
                lo = jnp.minimum(c[i2], c[j2]) if lo_need else None
                hi = jnp.maximum(c[i2], c[j2]) if hi_need else None
                if lo_need:
                    c[i2] = lo
                if hi_need:
                    c[j2] = hi
        run_ref[rows, :] = jnp.concatenate(c[:K], axis=1)
        return _

    jax.lax.fori_loop(0, HW // RC, _chunk, None)

    @pl.when(t == T_STEPS - 1)
    def _final():
        x = run_ref[...]
        q2 = jnp.sum(fv * fv, axis=1, keepdims=True)
        iota_l = jax.lax.broadcasted_iota(jnp.int32, (HW, KL), 1)
        big_i = jnp.int32(2 ** 30)

        vals = []
        for _ in range(K):
            m = jnp.min(x, axis=1, keepdims=True)
            pos = jnp.min(jnp.where(x == m, iota_l, big_i), axis=1, keepdims=True)
            x = jnp.where(iota_l == pos, BIG, x)
            vals.append(m)

        s = [jnp.sqrt(jnp.maximum(vv + q2, jnp.float32(0.0))) for vv in vals]

        pix_ref[...] = s[0]

        mx = jnp.max(s[0])
        iota_r = jax.lax.broadcasted_iota(jnp.int32, (HW, 1), 0)
        pos_r = jnp.min(jnp.where(s[0] == mx, iota_r, big_i))
        sel = [jnp.sum(jnp.where(iota_r == pos_r, si, jnp.float32(0.0)))
               for si in s]
        e = [jnp.exp(si - sel[K - 1]) for si in sel]
        denom = e[0]
        for ei in e[1:]:
            denom = denom + ei
        img = sel[0] * (jnp.float32(1.0) - e[0] / denom)
        b = pl.program_id(0)
        img_ref[pl.ds(b, 1), :] = img[None, None]


@jax.jit
def kernel(feature_batch, memory_bank):
    B, H, W, C_ = feature_batch.shape
    fv = feature_batch.reshape(B * H * W, C_)
    bank_t = jnp.pad(memory_bank, ((0, N_PAD - N_BANK), (0, 0))).T

    pix, img = pl.pallas_call(
        _scorer_body,
        grid=(B_IMGS, T_STEPS),
        in_specs=[
            pl.BlockSpec((HW, C), lambda b, t: (b, 0)),
            pl.BlockSpec((C, TB), lambda b, t: (0, t)),
        ],
        out_specs=[
            pl.BlockSpec((HW, 1), lambda b, t: (b, 0)),
            pl.BlockSpec((B_IMGS, 1), lambda b, t: (0, 0)),
        ],
        out_shape=[
            jax.ShapeDtypeStruct((B_IMGS * HW, 1), jnp.float32),
            jax.ShapeDtypeStruct((B_IMGS, 1), jnp.float32),
        ],
        scratch_shapes=[
            pltpu.VMEM((HW, TB), jnp.float32),
            pltpu.VMEM((HW, KL), jnp.float32),
        ],
        compiler_params=pltpu.CompilerParams(
            dimension_semantics=("arbitrary", "arbitrary"),
        ),
    )(fv, bank_t)

    pixel_scores = pix.reshape(B, 1, H, W)
    image_scores = img.reshape(B)
    return (pixel_scores, image_scores)


# speedup vs baseline: 1.1142x; 1.1142x over previous
import functools

import jax
import jax.numpy as jnp
from jax.experimental import pallas as pl
from jax.experimental.pallas import tpu as pltpu

B_IMGS = 2
HW = 1024
C = 128
N_BANK = 50000
G = 16
LANES = 128
TB = G * LANES
T_STEPS = (N_BANK + TB - 1) // TB
N_PAD = T_STEPS * TB
K = 9
BIG = 3.0e38


def _oems_pairs(n):
    pairs = []

    def merge(lo, n2, r):
        step = r * 2
        if step < n2:
            merge(lo, n2, step)
            merge(lo + r, n2, step)
            for i in range(lo + r, lo + n2 - r, step):
                pairs.append((i, i + r))
        else:
            pairs.append((lo, lo + r))

    def sort_range(lo, hi):
        if (hi - lo) >= 1:
            mid = lo + ((hi - lo) // 2)
            sort_range(lo, mid)
            sort_range(mid + 1, hi)
            merge(lo, hi - lo + 1, 1)

    sort_range(0, n - 1)
    return pairs


_SORT_PAIRS = _oems_pairs(G)


def _pruned_clean_ops(n, keep):
    stages = []
    d = n // 2
    while d >= 1:
        stages.append([(i, i + d)
                       for base in range(0, n, 2 * d)
                       for i in range(base, base + d)])
        d //= 2
    needed = set(range(keep))
    pruned = []
    for ops in reversed(stages):
        sp = []
        new_needed = set()
        for (i, j) in ops:
            lo, hi = i in needed, j in needed
            if lo or hi:
                sp.append((i, j, lo, hi))
                new_needed.add(i)
                new_needed.add(j)
        needed = new_needed
        pruned.append(sp)
    return list(reversed(pruned))


_CLEAN_OPS = _pruned_clean_ops(G, K)
KL = K * LANES


RC = 64


def _scorer_body(fv_ref, bankt_ref, pix_ref, img_ref, d_scr, run_ref):
    t = pl.program_id(1)
    fv = fv_ref[...]
    bankt = bankt_ref[...]

    m2 = jnp.sum(bankt * bankt, axis=0, keepdims=True)
    col = t * TB + jax.lax.broadcasted_iota(jnp.int32, (1, TB), 1)
    m2 = jnp.where(col < N_BANK, m2, BIG)

    d_scr[...] = jnp.dot(fv * jnp.float32(-2.0), bankt,
                         preferred_element_type=jnp.float32)

    @pl.when(t == 0)
    def _init():
        run_ref[...] = jnp.full((HW, KL), BIG, jnp.float32)

    def _chunk(i, _):
        rows = pl.ds(i * RC, RC)
        v = [d_scr[rows, j * LANES:(j + 1) * LANES] + m2[:, j * LANES:(j + 1) * LANES]
             for j in range(G)]
        for (i1, j1) in _SORT_PAIRS:
            lo = jnp.minimum(v[i1], v[j1])
            hi = jnp.maximum(v[i1], v[j1])
            v[i1] = lo
            v[j1] = hi
        r = [run_ref[rows, j * LANES:(j + 1) * LANES] for j in range(K)]
        c = ([jnp.minimum(r[j], v[G - 1 - j]) for j in range(K)]
             + [v[G - 1 - j] for j in range(K, G)])
        for stage in _CLEAN_OPS:
            for (i2, j2, lo_need, hi_need) in stage:
                lo = jnp.minimum(c[i2], c[j2]) if lo_need else None
                hi = jnp.maximum(c[i2], c[j2]) if hi_need else None
                if lo_need:
                    c[i2] = lo
                if hi_need:
                    c[j2] = hi
        run_ref[rows, :] = jnp.concatenate(c[:K], axis=1)
        return _

    jax.lax.fori_loop(0, HW // RC, _chunk, None)

    @pl.when(t == T_STEPS - 1)
    def _final():
        x = run_ref[...]
        q2 = jnp.sum(fv * fv, axis=1, keepdims=True)
        iota_l = jax.lax.broadcasted_iota(jnp.int32, (HW, KL), 1)
        big_i = jnp.int32(2 ** 30)

        vals = []
        for _ in range(K):
            m = jnp.min(x, axis=1, keepdims=True)
            pos = jnp.min(jnp.where(x == m, iota_l, big_i), axis=1, keepdims=True)
            x = jnp.where(iota_l == pos, BIG, x)
            vals.append(m)

        s = [jnp.sqrt(jnp.maximum(vv + q2, jnp.float32(0.0))) for vv in vals]

        pix_ref[...] = s[0]

        mx = jnp.max(s[0])
        iota_r = jax.lax.broadcasted_iota(jnp.int32, (HW, 1), 0)
        pos_r = jnp.min(jnp.where(s[0] == mx, iota_r, big_i))
        sel = [jnp.sum(jnp.where(iota_r == pos_r, si, jnp.float32(0.0)))
               for si in s]
        e = [jnp.exp(si - sel[K - 1]) for si in sel]
        denom = e[0]
        for ei in e[1:]:
            denom = denom + ei
        img = sel[0] * (jnp.float32(1.0) - e[0] / denom)
        b = pl.program_id(0)
        img_ref[pl.ds(b, 1), :] = img[None, None]


@jax.jit
def kernel(feature_batch, memory_bank):
    B, H, W, C_ = feature_batch.shape
    fv = feature_batch.reshape(B * H * W, C_)
    bank_t = jnp.pad(memory_bank, ((0, N_PAD - N_BANK), (0, 0))).T

    pix, img = pl.pallas_call(
        _scorer_body,
        grid=(B_IMGS, T_STEPS),
        in_specs=[
            pl.BlockSpec((HW, C), lambda b, t: (b, 0)),
            pl.BlockSpec((C, TB), lambda b, t: (0, t)),
        ],
        out_specs=[
            pl.BlockSpec((HW, 1), lambda b, t: (b, 0)),
            pl.BlockSpec((B_IMGS, 1), lambda b, t: (0, 0)),
        ],
        out_shape=[
            jax.ShapeDtypeStruct((B_IMGS * HW, 1), jnp.float32),
            jax.ShapeDtypeStruct((B_IMGS, 1), jnp.float32),
        ],
        scratch_shapes=[
            pltpu.VMEM((HW, TB), jnp.float32),
            pltpu.VMEM((HW, KL), jnp.float32),
        ],
        compiler_params=pltpu.CompilerParams(
            dimension_semantics=("arbitrary", "arbitrary"),
        ),
    )(fv, bank_t)

    pixel_scores = pix.reshape(B, 1, H, W)
    image_scores = img.reshape(B)
    return (pixel_scores, image_scores)


# speedup vs baseline: 1.2424x; 1.1151x over previous
import functools

import jax
import jax.numpy as jnp
from jax.experimental import pallas as pl
from jax.experimental.pallas import tpu as pltpu

B_IMGS = 2
HW = 1024
C = 128
N_BANK = 50000
G = 16
LANES = 128
TB = G * LANES
T_STEPS = (N_BANK + TB - 1) // TB
N_PAD = T_STEPS * TB
K = 9
BIG = 3.0e38


def _oems_pairs(n):
    pairs = []

    def merge(lo, n2, r):
        step = r * 2
        if step < n2:
            merge(lo, n2, step)
            merge(lo + r, n2, step)
            for i in range(lo + r, lo + n2 - r, step):
                pairs.append((i, i + r))
        else:
            pairs.append((lo, lo + r))

    def sort_range(lo, hi):
        if (hi - lo) >= 1:
            mid = lo + ((hi - lo) // 2)
            sort_range(lo, mid)
            sort_range(mid + 1, hi)
            merge(lo, hi - lo + 1, 1)

    sort_range(0, n - 1)
    return pairs


_SORT_PAIRS = _oems_pairs(G)


def _pruned_clean_ops(n, keep):
    stages = []
    d = n // 2
    while d >= 1:
        stages.append([(i, i + d)
                       for base in range(0, n, 2 * d)
                       for i in range(base, base + d)])
        d //= 2
    needed = set(range(keep))
    pruned = []
    for ops in reversed(stages):
        sp = []
        new_needed = set()
        for (i, j) in ops:
            lo, hi = i in needed, j in needed
            if lo or hi:
                sp.append((i, j, lo, hi))
                new_needed.add(i)
                new_needed.add(j)
        needed = new_needed
        pruned.append(sp)
    return list(reversed(pruned))


_CLEAN_OPS = _pruned_clean_ops(G, K)
KL = K * LANES


def _scorer_body(fv_ref, bankt_ref, pix_ref, img_ref, run_ref):
    t = pl.program_id(1)
    fv = fv_ref[...]
    bankt = bankt_ref[...]

    m2 = jnp.sum(bankt * bankt, axis=0, keepdims=True)
    col = t * TB + jax.lax.broadcasted_iota(jnp.int32, (1, TB), 1)
    m2 = jnp.where(col < N_BANK, m2, BIG)

    qm = jnp.dot(fv * jnp.float32(-2.0), bankt,
                 preferred_element_type=jnp.float32)
    d = qm + m2

    v = [d[:, j * LANES:(j + 1) * LANES] for j in range(G)]
    for (i, j) in _SORT_PAIRS:
        lo = jnp.minimum(v[i], v[j])
        hi = jnp.maximum(v[i], v[j])
        v[i] = lo
        v[j] = hi

    @pl.when(t == 0)
    def _init():
        run_ref[...] = jnp.concatenate(v[:K], axis=1)

    @pl.when(t > 0)
    def _merge():
        r = [run_ref[:, j * LANES:(j + 1) * LANES] for j in range(K)]
        c = ([jnp.minimum(r[j], v[G - 1 - j]) for j in range(K)]
             + [v[G - 1 - j] for j in range(K, G)])
        for stage in _CLEAN_OPS:
            for (i2, j2, lo_need, hi_need) in stage:
                lo = jnp.minimum(c[i2], c[j2]) if lo_need else None
                hi = jnp.maximum(c[i2], c[j2]) if hi_need else None
                if lo_need:
                    c[i2] = lo
                if hi_need:
                    c[j2] = hi
        run_ref[...] = jnp.concatenate(c[:K], axis=1)

    @pl.when(t == T_STEPS - 1)
    def _final():
        x = run_ref[...]
        q2 = jnp.sum(fv * fv, axis=1, keepdims=True)
        iota_l = jax.lax.broadcasted_iota(jnp.int32, (HW, KL), 1)
        big_i = jnp.int32(2 ** 30)

        vals = []
        for _ in range(K):
            m = jnp.min(x, axis=1, keepdims=True)
            pos = jnp.min(jnp.where(x == m, iota_l, big_i), axis=1, keepdims=True)
            x = jnp.where(iota_l == pos, BIG, x)
            vals.append(m)

        s = [jnp.sqrt(jnp.maximum(vv + q2, jnp.float32(0.0))) for vv in vals]

        pix_ref[...] = s[0]

        mx = jnp.max(s[0])
        iota_r = jax.lax.broadcasted_iota(jnp.int32, (HW, 1), 0)
        pos_r = jnp.min(jnp.where(s[0] == mx, iota_r, big_i))
        sel = [jnp.sum(jnp.where(iota_r == pos_r, si, jnp.float32(0.0)))
               for si in s]
        e = [jnp.exp(si - sel[K - 1]) for si in sel]
        denom = e[0]
        for ei in e[1:]:
            denom = denom + ei
        img = sel[0] * (jnp.float32(1.0) - e[0] / denom)
        b = pl.program_id(0)
        img_ref[pl.ds(b, 1), :] = img[None, None]


@jax.jit
def kernel(feature_batch, memory_bank):
    B, H, W, C_ = feature_batch.shape
    fv = feature_batch.reshape(B * H * W, C_)
    bank_t = jnp.pad(memory_bank, ((0, N_PAD - N_BANK), (0, 0))).T

    pix, img = pl.pallas_call(
        _scorer_body,
        grid=(B_IMGS, T_STEPS),
        in_specs=[
            pl.BlockSpec((HW, C), lambda b, t: (b, 0)),
            pl.BlockSpec((C, TB), lambda b, t: (0, t)),
        ],
        out_specs=[
            pl.BlockSpec((HW, 1), lambda b, t: (b, 0)),
            pl.BlockSpec((B_IMGS, 1), lambda b, t: (0, 0)),
        ],
        out_shape=[
            jax.ShapeDtypeStruct((B_IMGS * HW, 1), jnp.float32),
            jax.ShapeDtypeStruct((B_IMGS, 1), jnp.float32),
        ],
        scratch_shapes=[pltpu.VMEM((HW, KL), jnp.float32)],
        compiler_params=pltpu.CompilerParams(
            dimension_semantics=("arbitrary", "arbitrary"),
        ),
    )(fv, bank_t)

    pixel_scores = pix.reshape(B, 1, H, W)
    image_scores = img.reshape(B)
    return (pixel_scores, image_scores)


# speedup vs baseline: 1.4203x; 1.1431x over previous
import functools

import jax
import jax.numpy as jnp
from jax.experimental import pallas as pl
from jax.experimental.pallas import tpu as pltpu

B_IMGS = 2
HW = 1024
C = 128
N_BANK = 50000
G = 16
LANES = 128
TB = G * LANES
T_STEPS = (N_BANK + TB - 1) // TB
N_PAD = T_STEPS * TB
K = 9
BIG = 3.0e38


def _oems_pairs(n):
    pairs = []

    def merge(lo, n2, r):
        step = r * 2
        if step < n2:
            merge(lo, n2, step)
            merge(lo + r, n2, step)
            for i in range(lo + r, lo + n2 - r, step):
                pairs.append((i, i + r))
        else:
            pairs.append((lo, lo + r))

    def sort_range(lo, hi):
        if (hi - lo) >= 1:
            mid = lo + ((hi - lo) // 2)
            sort_range(lo, mid)
            sort_range(mid + 1, hi)
            merge(lo, hi - lo + 1, 1)

    sort_range(0, n - 1)
    return pairs


_SORT_PAIRS = _oems_pairs(G)


def _pruned_clean_ops(n, keep):
    stages = []
    d = n // 2
    while d >= 1:
        stages.append([(i, i + d)
                       for base in range(0, n, 2 * d)
                       for i in range(base, base + d)])
        d //= 2
    needed = set(range(keep))
    pruned = []
    for ops in reversed(stages):
        sp = []
        new_needed = set()
        for (i, j) in ops:
            lo, hi = i in needed, j in needed
            if lo or hi:
                sp.append((i, j, lo, hi))
                new_needed.add(i)
                new_needed.add(j)
        needed = new_needed
        pruned.append(sp)
    return list(reversed(pruned))


_CLEAN_OPS = _pruned_clean_ops(G, K)
KL = K * LANES


def _scorer_body(fv_ref, bank_ref, pix_ref, img_ref, run_ref):
    t = pl.program_id(1)
    fv = fv_ref[...]
    bank = bank_ref[...]

    m2 = jnp.sum(bank * bank, axis=1).reshape(1, TB)
    col = t * TB + jax.lax.broadcasted_iota(jnp.int32, (1, TB), 1)
    m2 = jnp.where(col < N_BANK, m2, BIG)

    qm = jax.lax.dot_general(fv * jnp.float32(-2.0), bank,
                             (((1,), (1,)), ((), ())),
                             preferred_element_type=jnp.float32)
    d = qm + m2

    v = [d[:, j * LANES:(j + 1) * LANES] for j in range(G)]
    for (i, j) in _SORT_PAIRS:
        lo = jnp.minimum(v[i], v[j])
        hi = jnp.maximum(v[i], v[j])
        v[i] = lo
        v[j] = hi

    @pl.when(t == 0)
    def _init():
        run_ref[...] = jnp.concatenate(v[:K], axis=1)

    @pl.when(t > 0)
    def _merge():
        r = [run_ref[:, j * LANES:(j + 1) * LANES] for j in range(K)]
        c = ([jnp.minimum(r[j], v[G - 1 - j]) for j in range(K)]
             + [v[G - 1 - j] for j in range(K, G)])
        for stage in _CLEAN_OPS:
            for (i2, j2, lo_need, hi_need) in stage:
                lo = jnp.minimum(c[i2], c[j2]) if lo_need else None
                hi = jnp.maximum(c[i2], c[j2]) if hi_need else None
                if lo_need:
                    c[i2] = lo
                if hi_need:
                    c[j2] = hi
        run_ref[...] = jnp.concatenate(c[:K], axis=1)

    @pl.when(t == T_STEPS - 1)
    def _final():
        x = run_ref[...]
        q2 = jnp.sum(fv * fv, axis=1, keepdims=True)
        iota_l = jax.lax.broadcasted_iota(jnp.int32, (HW, KL), 1)
        big_i = jnp.int32(2 ** 30)

        vals = []
        for _ in range(K):
            m = jnp.min(x, axis=1, keepdims=True)
            pos = jnp.min(jnp.where(x == m, iota_l, big_i), axis=1, keepdims=True)
            x = jnp.where(iota_l == pos, BIG, x)
            vals.append(m)

        s = [jnp.sqrt(jnp.maximum(vv + q2, jnp.float32(0.0))) for vv in vals]

        pix_ref[...] = s[0]

        mx = jnp.max(s[0])
        iota_r = jax.lax.broadcasted_iota(jnp.int32, (HW, 1), 0)
        pos_r = jnp.min(jnp.where(s[0] == mx, iota_r, big_i))
        sel = [jnp.sum(jnp.where(iota_r == pos_r, si, jnp.float32(0.0)))
               for si in s]
        e = [jnp.exp(si - sel[K - 1]) for si in sel]
        denom = e[0]
        for ei in e[1:]:
            denom = denom + ei
        img = sel[0] * (jnp.float32(1.0) - e[0] / denom)
        b = pl.program_id(0)
        img_ref[pl.ds(b, 1), :] = img[None, None]


@jax.jit
def kernel(feature_batch, memory_bank):
    B, H, W, C_ = feature_batch.shape
    fv = feature_batch.reshape(B * H * W, C_)
    bank_p = jnp.pad(memory_bank, ((0, N_PAD - N_BANK), (0, 0)))

    pix, img = pl.pallas_call(
        _scorer_body,
        grid=(B_IMGS, T_STEPS),
        in_specs=[
            pl.BlockSpec((HW, C), lambda b, t: (b, 0)),
            pl.BlockSpec((TB, C), lambda b, t: (t, 0)),
        ],
        out_specs=[
            pl.BlockSpec((HW, 1), lambda b, t: (b, 0)),
            pl.BlockSpec((B_IMGS, 1), lambda b, t: (0, 0)),
        ],
        out_shape=[
            jax.ShapeDtypeStruct((B_IMGS * HW, 1), jnp.float32),
            jax.ShapeDtypeStruct((B_IMGS, 1), jnp.float32),
        ],
        scratch_shapes=[pltpu.VMEM((HW, KL), jnp.float32)],
        compiler_params=pltpu.CompilerParams(
            dimension_semantics=("arbitrary", "arbitrary"),
        ),
    )(fv, bank_p)

    pixel_scores = pix.reshape(B, 1, H, W)
    image_scores = img.reshape(B)
    return (pixel_scores, image_scores)


# speedup vs baseline: 1.5172x; 1.0682x over previous
import functools

import jax
import jax.numpy as jnp
from jax.experimental import pallas as pl
from jax.experimental.pallas import tpu as pltpu

B_IMGS = 2
HW = 1024
C = 128
N_BANK = 50000
G = 16
LANES = 128
TB = G * LANES
T_STEPS = (N_BANK + TB - 1) // TB
N_PAD = T_STEPS * TB
K = 9
BIG = 3.0e38


def _oems_pairs(n):
    pairs = []

    def merge(lo, n2, r):
        step = r * 2
        if step < n2:
            merge(lo, n2, step)
            merge(lo + r, n2, step)
            for i in range(lo + r, lo + n2 - r, step):
                pairs.append((i, i + r))
        else:
            pairs.append((lo, lo + r))

    def sort_range(lo, hi):
        if (hi - lo) >= 1:
            mid = lo + ((hi - lo) // 2)
            sort_range(lo, mid)
            sort_range(mid + 1, hi)
            merge(lo, hi - lo + 1, 1)

    sort_range(0, n - 1)
    return pairs


def _oems_sort_pairs(n, offset):
    return [(i + offset, j + offset) for (i, j) in _oems_pairs(n)] if n > 1 else []


def _oem_merge_pairs(n):
    pairs = []

    def merge(lo, n2, r):
        step = r * 2
        if step < n2:
            merge(lo, n2, step)
            merge(lo + r, n2, step)
            for i in range(lo + r, lo + n2 - r, step):
                pairs.append((i, i + r))
        else:
            pairs.append((lo, lo + r))

    merge(0, n, 1)
    return pairs


def _build_merge_program(s):
    inf = [False] * K + [True] * (16 - K) + [False] * s + [True] * (16 - s)
    prog = []
    for (i, j) in _oems_sort_pairs(s, 16) + _oem_merge_pairs(32):
        if inf[j]:
            continue
        if inf[i]:
            prog.append(('mov', j, i))
            inf[i], inf[j] = False, True
            continue
        prog.append(('ce', i, j))
    needed = set(range(K))
    pruned = []
    for op in reversed(prog):
        if op[0] == 'ce':
            _, i, j = op
            lo_need, hi_need = i in needed, j in needed
            if not (lo_need or hi_need):
                continue
            pruned.append(('ce', i, j, lo_need, hi_need))
            needed.add(i)
            needed.add(j)
        else:
            _, src, dst = op
            if dst not in needed:
                continue
            pruned.append(op)
            needed.discard(dst)
            needed.add(src)
    pruned.reverse()
    return pruned


SUB = 8
_MERGE_PROG = _build_merge_program(SUB)
KL = K * LANES


def _apply_merge(r, new):
    slots = [None] * 32
    slots[:K] = r
    slots[16:16 + SUB] = new
    for op in _MERGE_PROG:
        if op[0] == 'mov':
            slots[op[2]] = slots[op[1]]
        else:
            _, i, j, lo_need, hi_need = op
            lo = jnp.minimum(slots[i], slots[j]) if lo_need else None
            hi = jnp.maximum(slots[i], slots[j]) if hi_need else None
            slots[i] = lo
            slots[j] = hi
    return slots[:K]


def _scorer_body(fv_ref, bank_ref, pix_ref, img_ref, run_ref):
    t = pl.program_id(1)
    fv = fv_ref[...]
    bank = bank_ref[...]

    m2 = jnp.sum(bank * bank, axis=1).reshape(1, TB)
    col = t * TB + jax.lax.broadcasted_iota(jnp.int32, (1, TB), 1)
    m2 = jnp.where(col < N_BANK, m2, BIG)

    qm = jax.lax.dot_general(fv * jnp.float32(-2.0), bank,
                             (((1,), (1,)), ((), ())),
                             preferred_element_type=jnp.float32)
    d = qm + m2

    @pl.when(t == 0)
    def _init():
        run_ref[...] = jnp.full((HW, KL), BIG, jnp.float32)

    v = [d[:, j * LANES:(j + 1) * LANES] for j in range(G)]
    r = [run_ref[:, j * LANES:(j + 1) * LANES] for j in range(K)]
    for half in range(G // SUB):
        r = _apply_merge(r, v[half * SUB:(half + 1) * SUB])
    run_ref[...] = jnp.concatenate(r, axis=1)

    @pl.when(t == T_STEPS - 1)
    def _final():
        x = run_ref[...]
        q2 = jnp.sum(fv * fv, axis=1, keepdims=True)
        iota_l = jax.lax.broadcasted_iota(jnp.int32, (HW, KL), 1)
        big_i = jnp.int32(2 ** 30)

        vals = []
        for _ in range(K):
            m = jnp.min(x, axis=1, keepdims=True)
            pos = jnp.min(jnp.where(x == m, iota_l, big_i), axis=1, keepdims=True)
            x = jnp.where(iota_l == pos, BIG, x)
            vals.append(m)

        s = [jnp.sqrt(jnp.maximum(vv + q2, jnp.float32(0.0))) for vv in vals]

        pix_ref[...] = s[0]

        mx = jnp.max(s[0])
        iota_r = jax.lax.broadcasted_iota(jnp.int32, (HW, 1), 0)
        pos_r = jnp.min(jnp.where(s[0] == mx, iota_r, big_i))
        sel = [jnp.sum(jnp.where(iota_r == pos_r, si, jnp.float32(0.0)))
               for si in s]
        e = [jnp.exp(si - sel[K - 1]) for si in sel]
        denom = e[0]
        for ei in e[1:]:
            denom = denom + ei
        img = sel[0] * (jnp.float32(1.0) - e[0] / denom)
        b = pl.program_id(0)
        img_ref[pl.ds(b, 1), :] = img[None, None]


@jax.jit
def kernel(feature_batch, memory_bank):
    B, H, W, C_ = feature_batch.shape
    fv = feature_batch.reshape(B * H * W, C_)
    bank_p = jnp.pad(memory_bank, ((0, N_PAD - N_BANK), (0, 0)))

    pix, img = pl.pallas_call(
        _scorer_body,
        grid=(B_IMGS, T_STEPS),
        in_specs=[
            pl.BlockSpec((HW, C), lambda b, t: (b, 0)),
            pl.BlockSpec((TB, C), lambda b, t: (t, 0)),
        ],
        out_specs=[
            pl.BlockSpec((HW, 1), lambda b, t: (b, 0)),
            pl.BlockSpec((B_IMGS, 1), lambda b, t: (0, 0)),
        ],
        out_shape=[
            jax.ShapeDtypeStruct((B_IMGS * HW, 1), jnp.float32),
            jax.ShapeDtypeStruct((B_IMGS, 1), jnp.float32),
        ],
        scratch_shapes=[pltpu.VMEM((HW, KL), jnp.float32)],
        compiler_params=pltpu.CompilerParams(
            dimension_semantics=("arbitrary", "arbitrary"),
        ),
    )(fv, bank_p)

    pixel_scores = pix.reshape(B, 1, H, W)
    image_scores = img.reshape(B)
    return (pixel_scores, image_scores)


# speedup vs baseline: 1.5614x; 1.0292x over previous
import functools

import jax
import jax.numpy as jnp
from jax.experimental import pallas as pl
from jax.experimental.pallas import tpu as pltpu

B_IMGS = 2
HW = 1024
C = 128
N_BANK = 50000
G = 16
LANES = 128
TB = G * LANES
T_STEPS = (N_BANK + TB - 1) // TB
N_PAD = T_STEPS * TB
K = 9
BIG = 3.0e38


def _oems_pairs(n):
    pairs = []

    def merge(lo, n2, r):
        step = r * 2
        if step < n2:
            merge(lo, n2, step)
            merge(lo + r, n2, step)
            for i in range(lo + r, lo + n2 - r, step):
                pairs.append((i, i + r))
        else:
            pairs.append((lo, lo + r))

    def sort_range(lo, hi):
        if (hi - lo) >= 1:
            mid = lo + ((hi - lo) // 2)
            sort_range(lo, mid)
            sort_range(mid + 1, hi)
            merge(lo, hi - lo + 1, 1)

    sort_range(0, n - 1)
    return pairs


def _oems_sort_pairs(n, offset):
    return [(i + offset, j + offset) for (i, j) in _oems_pairs(n)] if n > 1 else []


def _oem_merge_pairs(n):
    pairs = []

    def merge(lo, n2, r):
        step = r * 2
        if step < n2:
            merge(lo, n2, step)
            merge(lo + r, n2, step)
            for i in range(lo + r, lo + n2 - r, step):
                pairs.append((i, i + r))
        else:
            pairs.append((lo, lo + r))

    merge(0, n, 1)
    return pairs


def _build_merge_program(s):
    inf = [False] * K + [True] * (16 - K) + [False] * s + [True] * (16 - s)
    prog = []
    for (i, j) in _oems_sort_pairs(s, 16) + _oem_merge_pairs(32):
        if inf[j]:
            continue
        if inf[i]:
            prog.append(('mov', j, i))
            inf[i], inf[j] = False, True
            continue
        prog.append(('ce', i, j))
    needed = set(range(K))
    pruned = []
    for op in reversed(prog):
        if op[0] == 'ce':
            _, i, j = op
            lo_need, hi_need = i in needed, j in needed
            if not (lo_need or hi_need):
                continue
            pruned.append(('ce', i, j, lo_need, hi_need))
            needed.add(i)
            needed.add(j)
        else:
            _, src, dst = op
            if dst not in needed:
                continue
            pruned.append(op)
            needed.discard(dst)
            needed.add(src)
    pruned.reverse()
    return pruned


SUB = 8
_MERGE_PROG = _build_merge_program(SUB)
KL = K * LANES


def _apply_merge(r, new):
    slots = [None] * 32
    slots[:K] = r
    slots[16:16 + SUB] = new
    for op in _MERGE_PROG:
        if op[0] == 'mov':
            slots[op[2]] = slots[op[1]]
        else:
            _, i, j, lo_need, hi_need = op
            lo = jnp.minimum(slots[i], slots[j]) if lo_need else None
            hi = jnp.maximum(slots[i], slots[j]) if hi_need else None
            slots[i] = lo
            slots[j] = hi
    return slots[:K]


QR = B_IMGS * HW


def _scorer_body(fv_ref, bank_ref, pix_ref, img_ref, run_ref):
    t = pl.program_id(0)
    fv = fv_ref[...]
    bank = bank_ref[...]

    m2 = jnp.sum(bank * bank, axis=1).reshape(1, TB)
    col = t * TB + jax.lax.broadcasted_iota(jnp.int32, (1, TB), 1)
    m2 = jnp.where(col < N_BANK, m2, BIG)

    qm = jax.lax.dot_general(fv * jnp.float32(-2.0), bank,
                             (((1,), (1,)), ((), ())),
                             preferred_element_type=jnp.float32)
    d = qm + m2

    @pl.when(t == 0)
    def _init():
        run_ref[...] = jnp.full((QR, KL), BIG, jnp.float32)

    v = [d[:, j * LANES:(j + 1) * LANES] for j in range(G)]
    r = [run_ref[:, j * LANES:(j + 1) * LANES] for j in range(K)]
    for half in range(G // SUB):
        r = _apply_merge(r, v[half * SUB:(half + 1) * SUB])
    run_ref[...] = jnp.concatenate(r, axis=1)

    @pl.when(t == T_STEPS - 1)
    def _final():
        x = run_ref[...]
        q2 = jnp.sum(fv * fv, axis=1, keepdims=True)
        iota_l = jax.lax.broadcasted_iota(jnp.int32, (QR, KL), 1)
        big_i = jnp.int32(2 ** 30)

        vals = []
        for _ in range(K):
            m = jnp.min(x, axis=1, keepdims=True)
            pos = jnp.min(jnp.where(x == m, iota_l, big_i), axis=1, keepdims=True)
            x = jnp.where(iota_l == pos, BIG, x)
            vals.append(m)

        s = [jnp.sqrt(jnp.maximum(vv + q2, jnp.float32(0.0))) for vv in vals]

        pix_ref[...] = s[0]

        iota_r = jax.lax.broadcasted_iota(jnp.int32, (HW, 1), 0)
        for bb in range(B_IMGS):
            sb = [si[bb * HW:(bb + 1) * HW, :] for si in s]
            mx = jnp.max(sb[0])
            pos_r = jnp.min(jnp.where(sb[0] == mx, iota_r, big_i))
            sel = [jnp.sum(jnp.where(iota_r == pos_r, si, jnp.float32(0.0)))
                   for si in sb]
            e = [jnp.exp(si - sel[K - 1]) for si in sel]
            denom = e[0]
            for ei in e[1:]:
                denom = denom + ei
            img = sel[0] * (jnp.float32(1.0) - e[0] / denom)
            img_ref[bb:bb + 1, :] = img[None, None]


@jax.jit
def kernel(feature_batch, memory_bank):
    B, H, W, C_ = feature_batch.shape
    fv = feature_batch.reshape(B * H * W, C_)
    bank_p = jnp.pad(memory_bank, ((0, N_PAD - N_BANK), (0, 0)))

    pix, img = pl.pallas_call(
        _scorer_body,
        grid=(T_STEPS,),
        in_specs=[
            pl.BlockSpec((QR, C), lambda t: (0, 0)),
            pl.BlockSpec((TB, C), lambda t: (t, 0)),
        ],
        out_specs=[
            pl.BlockSpec((QR, 1), lambda t: (0, 0)),
            pl.BlockSpec((B_IMGS, 1), lambda t: (0, 0)),
        ],
        out_shape=[
            jax.ShapeDtypeStruct((QR, 1), jnp.float32),
            jax.ShapeDtypeStruct((B_IMGS, 1), jnp.float32),
        ],
        scratch_shapes=[pltpu.VMEM((QR, KL), jnp.float32)],
        compiler_params=pltpu.CompilerParams(
            dimension_semantics=("arbitrary",),
        ),
    )(fv, bank_p)

    pixel_scores = pix.reshape(B, 1, H, W)
    image_scores = img.reshape(B)
    return (pixel_scores, image_scores)


# speedup vs baseline: 1.6869x; 1.0803x over previous
import functools

import jax
import jax.numpy as jnp
from jax.experimental import pallas as pl
from jax.experimental.pallas import tpu as pltpu

B_IMGS = 2
HW = 1024
C = 128
N_BANK = 50000
G = 16
LANES = 128
TB = G * LANES
T_STEPS = (N_BANK + TB - 1) // TB
N_PAD = T_STEPS * TB
K = 9
BIG = 3.0e38


def _oems_pairs(n):
    pairs = []

    def merge(lo, n2, r):
        step = r * 2
        if step < n2:
            merge(lo, n2, step)
            merge(lo + r, n2, step)
            for i in range(lo + r, lo + n2 - r, step):
                pairs.append((i, i + r))
        else:
            pairs.append((lo, lo + r))

    def sort_range(lo, hi):
        if (hi - lo) >= 1:
            mid = lo + ((hi - lo) // 2)
            sort_range(lo, mid)
            sort_range(mid + 1, hi)
            merge(lo, hi - lo + 1, 1)

    sort_range(0, n - 1)
    return pairs


def _oems_sort_pairs(n, offset):
    return [(i + offset, j + offset) for (i, j) in _oems_pairs(n)] if n > 1 else []


def _oem_merge_pairs(n):
    pairs = []

    def merge(lo, n2, r):
        step = r * 2
        if step < n2:
            merge(lo, n2, step)
            merge(lo + r, n2, step)
            for i in range(lo + r, lo + n2 - r, step):
                pairs.append((i, i + r))
        else:
            pairs.append((lo, lo + r))

    merge(0, n, 1)
    return pairs


def _build_merge_program(s):
    inf = [False] * K + [True] * (16 - K) + [False] * s + [True] * (16 - s)
    prog = []
    for (i, j) in _oems_sort_pairs(s, 16) + _oem_merge_pairs(32):
        if inf[j]:
            continue
        if inf[i]:
            prog.append(('mov', j, i))
            inf[i], inf[j] = False, True
            continue
        prog.append(('ce', i, j))
    needed = set(range(K))
    pruned = []
    for op in reversed(prog):
        if op[0] == 'ce':
            _, i, j = op
            lo_need, hi_need = i in needed, j in needed
            if not (lo_need or hi_need):
                continue
            pruned.append(('ce', i, j, lo_need, hi_need))
            needed.add(i)
            needed.add(j)
        else:
            _, src, dst = op
            if dst not in needed:
                continue
            pruned.append(op)
            needed.discard(dst)
            needed.add(src)
    pruned.reverse()
    return pruned


SUB = 8
_MERGE_PROG = _build_merge_program(SUB)
KL = K * LANES


def _apply_merge(r, new):
    slots = [None] * 32
    slots[:K] = r
    slots[16:16 + SUB] = new
    for op in _MERGE_PROG:
        if op[0] == 'mov':
            slots[op[2]] = slots[op[1]]
        else:
            _, i, j, lo_need, hi_need = op
            lo = jnp.minimum(slots[i], slots[j]) if lo_need else None
            hi = jnp.maximum(slots[i], slots[j]) if hi_need else None
            slots[i] = lo
            slots[j] = hi
    return slots[:K]


QR = B_IMGS * HW


def _scorer_body(fv_ref, bank_ref, pix_ref, img_ref, run_ref):
    t = pl.program_id(0)
    fv = fv_ref[...]
    bank = bank_ref[...]

    row = t * TB + jax.lax.broadcasted_iota(jnp.int32, (TB, 1), 0)
    bank = jnp.where(row < N_BANK, bank, jnp.float32(0.0))

    m2 = jnp.sum(bank * bank, axis=1).reshape(1, TB)
    col = t * TB + jax.lax.broadcasted_iota(jnp.int32, (1, TB), 1)
    m2 = jnp.where(col < N_BANK, m2, BIG)

    qm = jax.lax.dot_general(fv * jnp.float32(-2.0), bank,
                             (((1,), (1,)), ((), ())),
                             preferred_element_type=jnp.float32)
    d = qm + m2

    @pl.when(t == 0)
    def _init():
        run_ref[...] = jnp.full((QR, KL), BIG, jnp.float32)

    v = [d[:, j * LANES:(j + 1) * LANES] for j in range(G)]
    r = [run_ref[:, j * LANES:(j + 1) * LANES] for j in range(K)]
    for half in range(G // SUB):
        r = _apply_merge(r, v[half * SUB:(half + 1) * SUB])
    run_ref[...] = jnp.concatenate(r, axis=1)

    @pl.when(t == T_STEPS - 1)
    def _final():
        x = run_ref[...]
        q2 = jnp.sum(fv * fv, axis=1, keepdims=True)
        iota_l = jax.lax.broadcasted_iota(jnp.int32, (QR, KL), 1)
        big_i = jnp.int32(2 ** 30)

        vals = []
        for _ in range(K):
            m = jnp.min(x, axis=1, keepdims=True)
            pos = jnp.min(jnp.where(x == m, iota_l, big_i), axis=1, keepdims=True)
            x = jnp.where(iota_l == pos, BIG, x)
            vals.append(m)

        s = [jnp.sqrt(jnp.maximum(vv + q2, jnp.float32(0.0))) for vv in vals]

        pix_ref[...] = s[0]

        iota_r = jax.lax.broadcasted_iota(jnp.int32, (HW, 1), 0)
        for bb in range(B_IMGS):
            sb = [si[bb * HW:(bb + 1) * HW, :] for si in s]
            mx = jnp.max(sb[0])
            pos_r = jnp.min(jnp.where(sb[0] == mx, iota_r, big_i))
            sel = [jnp.sum(jnp.where(iota_r == pos_r, si, jnp.float32(0.0)))
                   for si in sb]
            e = [jnp.exp(si - sel[K - 1]) for si in sel]
            denom = e[0]
            for ei in e[1:]:
                denom = denom + ei
            img = sel[0] * (jnp.float32(1.0) - e[0] / denom)
            img_ref[bb:bb + 1, :] = img[None, None]


@jax.jit
def kernel(feature_batch, memory_bank):
    B, H, W, C_ = feature_batch.shape
    fv = feature_batch.reshape(B * H * W, C_)

    pix, img = pl.pallas_call(
        _scorer_body,
        grid=(T_STEPS,),
        in_specs=[
            pl.BlockSpec((QR, C), lambda t: (0, 0)),
            pl.BlockSpec((TB, C), lambda t: (t, 0)),
        ],
        out_specs=[
            pl.BlockSpec((QR, 1), lambda t: (0, 0)),
            pl.BlockSpec((B_IMGS, 1), lambda t: (0, 0)),
        ],
        out_shape=[
            jax.ShapeDtypeStruct((QR, 1), jnp.float32),
            jax.ShapeDtypeStruct((B_IMGS, 1), jnp.float32),
        ],
        scratch_shapes=[pltpu.VMEM((QR, KL), jnp.float32)],
        compiler_params=pltpu.CompilerParams(
            dimension_semantics=("arbitrary",),
        ),
    )(fv, memory_bank)

    pixel_scores = pix.reshape(B, 1, H, W)
    image_scores = img.reshape(B)
    return (pixel_scores, image_scores)


# speedup vs baseline: 1.6872x; 1.0002x over previous
import functools

import jax
import jax.numpy as jnp
from jax.experimental import pallas as pl
from jax.experimental.pallas import tpu as pltpu

B_IMGS = 2
HW = 1024
C = 128
N_BANK = 50000
G = 16
LANES = 128
TB = G * LANES
T_STEPS = (N_BANK + TB - 1) // TB
N_PAD = T_STEPS * TB
K = 9
BIG = 3.0e38


def _oems_pairs(n):
    pairs = []

    def merge(lo, n2, r):
        step = r * 2
        if step < n2:
            merge(lo, n2, step)
            merge(lo + r, n2, step)
            for i in range(lo + r, lo + n2 - r, step):
                pairs.append((i, i + r))
        else:
            pairs.append((lo, lo + r))

    def sort_range(lo, hi):
        if (hi - lo) >= 1:
            mid = lo + ((hi - lo) // 2)
            sort_range(lo, mid)
            sort_range(mid + 1, hi)
            merge(lo, hi - lo + 1, 1)

    sort_range(0, n - 1)
    return pairs


def _oems_sort_pairs(n, offset):
    return [(i + offset, j + offset) for (i, j) in _oems_pairs(n)] if n > 1 else []


def _oem_merge_pairs(n):
    pairs = []

    def merge(lo, n2, r):
        step = r * 2
        if step < n2:
            merge(lo, n2, step)
            merge(lo + r, n2, step)
            for i in range(lo + r, lo + n2 - r, step):
                pairs.append((i, i + r))
        else:
            pairs.append((lo, lo + r))

    merge(0, n, 1)
    return pairs


def _build_merge_program(s):
    inf = [False] * K + [True] * (16 - K) + [False] * s + [True] * (16 - s)
    prog = []
    for (i, j) in _oems_sort_pairs(s, 16) + _oem_merge_pairs(32):
        if inf[j]:
            continue
        if inf[i]:
            prog.append(('mov', j, i))
            inf[i], inf[j] = False, True
            continue
        prog.append(('ce', i, j))
    needed = set(range(K))
    pruned = []
    for op in reversed(prog):
        if op[0] == 'ce':
            _, i, j = op
            lo_need, hi_need = i in needed, j in needed
            if not (lo_need or hi_need):
                continue
            pruned.append(('ce', i, j, lo_need, hi_need))
            needed.add(i)
            needed.add(j)
        else:
            _, src, dst = op
            if dst not in needed:
                continue
            pruned.append(op)
            needed.discard(dst)
            needed.add(src)
    pruned.reverse()
    return pruned


SUB = 8
_MERGE_PROG = _build_merge_program(SUB)
KL = K * LANES


def _apply_merge(r, new):
    slots = [None] * 32
    slots[:K] = r
    slots[16:16 + SUB] = new
    for op in _MERGE_PROG:
        if op[0] == 'mov':
            slots[op[2]] = slots[op[1]]
        else:
            _, i, j, lo_need, hi_need = op
            lo = jnp.minimum(slots[i], slots[j]) if lo_need else None
            hi = jnp.maximum(slots[i], slots[j]) if hi_need else None
            slots[i] = lo
            slots[j] = hi
    return slots[:K]


QR = B_IMGS * HW


def _scorer_body(fv_ref, bank_ref, pix_ref, img_ref, run_ref):
    t = pl.program_id(0)
    fv = fv_ref[...]
    bank = bank_ref[...]

    row = t * TB + jax.lax.broadcasted_iota(jnp.int32, (TB, 1), 0)
    bank = jnp.where(row < N_BANK, bank, jnp.float32(0.0))

    m2 = jnp.sum(bank * bank, axis=1).reshape(1, TB)
    col = t * TB + jax.lax.broadcasted_iota(jnp.int32, (1, TB), 1)
    m2 = jnp.where(col < N_BANK, m2, BIG)

    qm = jax.lax.dot_general(fv * jnp.float32(-2.0), bank,
                             (((1,), (1,)), ((), ())),
                             preferred_element_type=jnp.float32,
                             precision=jax.lax.Precision.DEFAULT)
    d = qm + m2

    @pl.when(t == 0)
    def _init():
        run_ref[...] = jnp.full((QR, KL), BIG, jnp.float32)

    v = [d[:, j * LANES:(j + 1) * LANES] for j in range(G)]
    r = [run_ref[:, j * LANES:(j + 1) * LANES] for j in range(K)]
    for half in range(G // SUB):
        r = _apply_merge(r, v[half * SUB:(half + 1) * SUB])
    run_ref[...] = jnp.concatenate(r, axis=1)

    @pl.when(t == T_STEPS - 1)
    def _final():
        x = run_ref[...]
        q2 = jnp.sum(fv * fv, axis=1, keepdims=True)
        iota_l = jax.lax.broadcasted_iota(jnp.int32, (QR, KL), 1)
        big_i = jnp.int32(2 ** 30)

        vals = []
        for _ in range(K):
            m = jnp.min(x, axis=1, keepdims=True)
            pos = jnp.min(jnp.where(x == m, iota_l, big_i), axis=1, keepdims=True)
            x = jnp.where(iota_l == pos, BIG, x)
            vals.append(m)

        s = [jnp.sqrt(jnp.maximum(vv + q2, jnp.float32(0.0))) for vv in vals]

        pix_ref[...] = s[0]

        iota_r = jax.lax.broadcasted_iota(jnp.int32, (HW, 1), 0)
        for bb in range(B_IMGS):
            sb = [si[bb * HW:(bb + 1) * HW, :] for si in s]
            mx = jnp.max(sb[0])
            pos_r = jnp.min(jnp.where(sb[0] == mx, iota_r, big_i))
            sel = [jnp.sum(jnp.where(iota_r == pos_r, si, jnp.float32(0.0)))
                   for si in sb]
            e = [jnp.exp(si - sel[K - 1]) for si in sel]
            denom = e[0]
            for ei in e[1:]:
                denom = denom + ei
            img = sel[0] * (jnp.float32(1.0) - e[0] / denom)
            img_ref[bb:bb + 1, :] = img[None, None]


@jax.jit
def kernel(feature_batch, memory_bank):
    B, H, W, C_ = feature_batch.shape
    fv = feature_batch.reshape(B * H * W, C_)

    pix, img = pl.pallas_call(
        _scorer_body,
        grid=(T_STEPS,),
        in_specs=[
            pl.BlockSpec((QR, C), lambda t: (0, 0)),
            pl.BlockSpec((TB, C), lambda t: (t, 0)),
        ],
        out_specs=[
            pl.BlockSpec((QR, 1), lambda t: (0, 0)),
            pl.BlockSpec((B_IMGS, 1), lambda t: (0, 0)),
        ],
        out_shape=[
            jax.ShapeDtypeStruct((QR, 1), jnp.float32),
            jax.ShapeDtypeStruct((B_IMGS, 1), jnp.float32),
        ],
        scratch_shapes=[pltpu.VMEM((QR, KL), jnp.float32)],
        compiler_params=pltpu.CompilerParams(
            dimension_semantics=("arbitrary",),
        ),
    )(fv, memory_bank)

    pixel_scores = pix.reshape(B, 1, H, W)
    image_scores = img.reshape(B)
    return (pixel_scores, image_scores)


# speedup vs baseline: 1.7918x; 1.0620x over previous
import functools

import jax
import jax.numpy as jnp
from jax.experimental import pallas as pl
from jax.experimental.pallas import tpu as pltpu

B_IMGS = 2
HW = 1024
C = 128
N_BANK = 50000
G = 16
LANES = 128
TB = G * LANES
T_STEPS = (N_BANK + TB - 1) // TB
N_PAD = T_STEPS * TB
K = 9
BIG = 3.0e38


def _oems_pairs(n):
    pairs = []

    def merge(lo, n2, r):
        step = r * 2
        if step < n2:
            merge(lo, n2, step)
            merge(lo + r, n2, step)
            for i in range(lo + r, lo + n2 - r, step):
                pairs.append((i, i + r))
        else:
            pairs.append((lo, lo + r))

    def sort_range(lo, hi):
        if (hi - lo) >= 1:
            mid = lo + ((hi - lo) // 2)
            sort_range(lo, mid)
            sort_range(mid + 1, hi)
            merge(lo, hi - lo + 1, 1)

    sort_range(0, n - 1)
    return pairs


def _oems_sort_pairs(n, offset):
    return [(i + offset, j + offset) for (i, j) in _oems_pairs(n)] if n > 1 else []


def _oem_merge_pairs(n):
    pairs = []

    def merge(lo, n2, r):
        step = r * 2
        if step < n2:
            merge(lo, n2, step)
            merge(lo + r, n2, step)
            for i in range(lo + r, lo + n2 - r, step):
                pairs.append((i, i + r))
        else:
            pairs.append((lo, lo + r))

    merge(0, n, 1)
    return pairs


def _build_merge_program(s):
    inf = [False] * K + [True] * (16 - K) + [False] * s + [True] * (16 - s)
    prog = []
    for (i, j) in _oems_sort_pairs(s, 16) + _oem_merge_pairs(32):
        if inf[j]:
            continue
        if inf[i]:
            prog.append(('mov', j, i))
            inf[i], inf[j] = False, True
            continue
        prog.append(('ce', i, j))
    needed = set(range(K))
    pruned = []
    for op in reversed(prog):
        if op[0] == 'ce':
            _, i, j = op
            lo_need, hi_need = i in needed, j in needed
            if not (lo_need or hi_need):
                continue
            pruned.append(('ce', i, j, lo_need, hi_need))
            needed.add(i)
            needed.add(j)
        else:
            _, src, dst = op
            if dst not in needed:
                continue
            pruned.append(op)
            needed.discard(dst)
            needed.add(src)
    pruned.reverse()
    return pruned


SUB = 8
_MERGE_PROG = _build_merge_program(SUB)
KL = K * LANES


def _apply_merge(r, new):
    slots = [None] * 32
    slots[:K] = r
    slots[16:16 + SUB] = new
    for op in _MERGE_PROG:
        if op[0] == 'mov':
            slots[op[2]] = slots[op[1]]
        else:
            _, i, j, lo_need, hi_need = op
            lo = jnp.minimum(slots[i], slots[j]) if lo_need else None
            hi = jnp.maximum(slots[i], slots[j]) if hi_need else None
            slots[i] = lo
            slots[j] = hi
    return slots[:K]


QR = B_IMGS * HW


def _scorer_body(fv_ref, bank_ref, pix_ref, img_ref, run_ref):
    t = pl.program_id(0)
    fv = fv_ref[...]
    bank = bank_ref[...]

    row = t * TB + jax.lax.broadcasted_iota(jnp.int32, (TB, 1), 0)
    bank = jnp.where(row < N_BANK, bank, jnp.float32(0.0))

    m2 = jnp.sum(bank * bank, axis=1).reshape(1, TB)
    col = t * TB + jax.lax.broadcasted_iota(jnp.int32, (1, TB), 1)
    m2 = jnp.where(col < N_BANK, m2, BIG)

    qm = jax.lax.dot_general(fv * jnp.float32(-2.0), bank,
                             (((1,), (1,)), ((), ())),
                             preferred_element_type=jnp.float32)
    d = qm + m2

    @pl.when(t == 0)
    def _init():
        run_ref[...] = jnp.full((QR, KL), BIG, jnp.float32)

    v = [d[:, j * LANES:(j + 1) * LANES] for j in range(G)]
    r = [run_ref[:, j * LANES:(j + 1) * LANES] for j in range(K)]
    for half in range(G // SUB):
        r = _apply_merge(r, v[half * SUB:(half + 1) * SUB])
    run_ref[...] = jnp.concatenate(r, axis=1)

    @pl.when(t == T_STEPS - 1)
    def _final():
        big_i = jnp.int32(2 ** 30)
        q2 = jnp.sum(fv * fv, axis=1, keepdims=True)

        d0 = jnp.min(r[0], axis=1, keepdims=True)
        s0 = jnp.sqrt(jnp.maximum(d0 + q2, jnp.float32(0.0)))
        pix_ref[...] = s0

        iota_r = jax.lax.broadcasted_iota(jnp.int32, (HW, 1), 0)
        iota_l = jax.lax.broadcasted_iota(jnp.int32, (1, KL), 1)
        for bb in range(B_IMGS):
            s0b = s0[bb * HW:(bb + 1) * HW, :]
            q2b = q2[bb * HW:(bb + 1) * HW, :]
            mx = jnp.max(s0b)
            pos_r = jnp.min(jnp.where(s0b == mx, iota_r, big_i))
            rowmask = iota_r == pos_r
            q2row = jnp.sum(jnp.where(rowmask, q2b, jnp.float32(0.0)))
            cand = [jnp.sum(jnp.where(rowmask, r[j][bb * HW:(bb + 1) * HW, :],
                                      jnp.float32(0.0)), axis=0, keepdims=True)
                    for j in range(K)]
            x = jnp.concatenate(cand, axis=1)
            vals = []
            for _ in range(K):
                m = jnp.min(x)
                p = jnp.min(jnp.where(x == m, iota_l, big_i))
                x = jnp.where(iota_l == p, BIG, x)
                vals.append(m)
            s = [jnp.sqrt(jnp.maximum(vv + q2row, jnp.float32(0.0)))
                 for vv in vals]
            e = [jnp.exp(si - s[K - 1]) for si in s]
            denom = e[0]
            for ei in e[1:]:
                denom = denom + ei
            img = s[0] * (jnp.float32(1.0) - e[0] / denom)
            img_ref[bb:bb + 1, :] = img[None, None]


@jax.jit
def kernel(feature_batch, memory_bank):
    B, H, W, C_ = feature_batch.shape
    fv = feature_batch.reshape(B * H * W, C_)

    pix, img = pl.pallas_call(
        _scorer_body,
        grid=(T_STEPS,),
        in_specs=[
            pl.BlockSpec((QR, C), lambda t: (0, 0)),
            pl.BlockSpec((TB, C), lambda t: (t, 0)),
        ],
        out_specs=[
            pl.BlockSpec((QR, 1), lambda t: (0, 0)),
            pl.BlockSpec((B_IMGS, 1), lambda t: (0, 0)),
        ],
        out_shape=[
            jax.ShapeDtypeStruct((QR, 1), jnp.float32),
            jax.ShapeDtypeStruct((B_IMGS, 1), jnp.float32),
        ],
        scratch_shapes=[pltpu.VMEM((QR, KL), jnp.float32)],
        compiler_params=pltpu.CompilerParams(
            dimension_semantics=("arbitrary",),
        ),
    )(fv, memory_bank)

    pixel_scores = pix.reshape(B, 1, H, W)
    image_scores = img.reshape(B)
    return (pixel_scores, image_scores)
